# baseline (speedup 1.0000x reference)
import functools

import jax
import jax.numpy as jnp
from jax import lax
from jax.experimental import pallas as pl
from jax.experimental.pallas import tpu as pltpu
from jax.experimental.pallas import tpu_sc as plsc

B = 16384
D = 128
L = 16
NC = 2
NS = 16
NW = NC * NS
BPW = B // NW
CHUNK = 128
NCHUNK = BPW // CHUNK


def _sc_body(uid_hbm, mid_hbm, uemb_hbm, memb_hbm, ubias_hbm, mbias_hbm,
             out_hbm, uid_v, mid_v, urows_v, mrows_v, ubias_v, mbias_v,
             out_v, sem0, sem1, sem2, sem3):
    wid = lax.axis_index("s") * NC + lax.axis_index("c")
    base = wid * BPW
    lanes = lax.broadcasted_iota(jnp.int32, (L,), 0)

    for j in range(NCHUNK):
        pltpu.sync_copy(uid_hbm.at[pl.ds(base + j * CHUNK, CHUNK)],
                        uid_v.at[j])
        pltpu.sync_copy(mid_hbm.at[pl.ds(base + j * CHUNK, CHUNK)],
                        mid_v.at[j])

    def do_chunk(j):
        cu = pltpu.async_copy(uemb_hbm.at[uid_v.at[j]], urows_v, sem0)
        cm = pltpu.async_copy(memb_hbm.at[mid_v.at[j]], mrows_v, sem1)
        cub = pltpu.async_copy(ubias_hbm.at[uid_v.at[j]], ubias_v, sem2)
        cmb = pltpu.async_copy(mbias_hbm.at[mid_v.at[j]], mbias_v, sem3)
        cu.wait()
        cm.wait()
        cub.wait()
        cmb.wait()

        def group(g, _):
            vec = jnp.zeros((L,), jnp.float32)
            for i in range(L):
                r = g * L + i
                acc = urows_v[r, pl.ds(0, L)] * mrows_v[r, pl.ds(0, L)]
                for k in range(1, D // L):
                    acc = acc + (urows_v[r, pl.ds(k * L, L)]
                                 * mrows_v[r, pl.ds(k * L, L)])
                s = jnp.sum(acc)
                vec = jnp.where(lanes == i, s, vec)
            vec = vec + ubias_v[pl.ds(g * L, L)] + mbias_v[pl.ds(g * L, L)]
            out_v[pl.ds(j * CHUNK + g * L, L)] = vec
            return 0

        lax.fori_loop(0, CHUNK // L, group, 0)

    for j in range(NCHUNK):
        do_chunk(j)

    pltpu.sync_copy(out_v, out_hbm.at[pl.ds(base, BPW)])


@jax.jit
def _run(user_ids, movie_ids, user_emb, movie_emb, user_bias, movie_bias):
    mesh = plsc.VectorSubcoreMesh(core_axis_name="c", subcore_axis_name="s")
    f = functools.partial(
        pl.kernel,
        mesh=mesh,
        compiler_params=pltpu.CompilerParams(needs_layout_passes=False),
        out_type=jax.ShapeDtypeStruct((B,), jnp.float32),
        scratch_types=[
            pltpu.VMEM((NCHUNK, CHUNK), jnp.int32),
            pltpu.VMEM((NCHUNK, CHUNK), jnp.int32),
            pltpu.VMEM((CHUNK, D), jnp.float32),
            pltpu.VMEM((CHUNK, D), jnp.float32),
            pltpu.VMEM((CHUNK,), jnp.float32),
            pltpu.VMEM((CHUNK,), jnp.float32),
            pltpu.VMEM((BPW,), jnp.float32),
            pltpu.SemaphoreType.DMA,
            pltpu.SemaphoreType.DMA,
            pltpu.SemaphoreType.DMA,
            pltpu.SemaphoreType.DMA,
        ],
    )(_sc_body)
    return f(user_ids, movie_ids, user_emb, movie_emb, user_bias, movie_bias)


def kernel(user_ids, movie_ids, user_emb, movie_emb, user_bias, movie_bias):
    return _run(user_ids.astype(jnp.int32), movie_ids.astype(jnp.int32),
                user_emb, movie_emb,
                user_bias.reshape(-1), movie_bias.reshape(-1))


# speedup vs baseline: 1.0698x; 1.0698x over previous
import functools

import jax
import jax.numpy as jnp
from jax import lax
from jax.experimental import pallas as pl
from jax.experimental.pallas import tpu as pltpu
from jax.experimental.pallas import tpu_sc as plsc

B = 16384
D = 128
L = 16
NC = 2
NS = 16
NW = NC * NS
BPW = B // NW
CHUNK = 128
NCHUNK = BPW // CHUNK


def _sc_body(uid_hbm, mid_hbm, uemb_hbm, memb_hbm, ubias_hbm, mbias_hbm,
             out_hbm, uid_v, mid_v, u0, u1, m0, m1, ub_v, mb_v,
             out_v, su0, su1, sm0, sm1, sb):
    wid = lax.axis_index("s") * NC + lax.axis_index("c")
    base = wid * BPW
    lanes = lax.broadcasted_iota(jnp.int32, (L,), 0)
    ubufs = (u0, u1)
    mbufs = (m0, m1)
    usems = (su0, su1)
    msems = (sm0, sm1)

    for j in range(NCHUNK):
        pltpu.sync_copy(uid_hbm.at[pl.ds(base + j * CHUNK, CHUNK)],
                        uid_v.at[j])
        pltpu.sync_copy(mid_hbm.at[pl.ds(base + j * CHUNK, CHUNK)],
                        mid_v.at[j])

    bias_copies = []
    for j in range(NCHUNK):
        bias_copies.append(
            pltpu.async_copy(ubias_hbm.at[uid_v.at[j]], ub_v.at[j], sb))
        bias_copies.append(
            pltpu.async_copy(mbias_hbm.at[mid_v.at[j]], mb_v.at[j], sb))

    def start(j):
        cu = pltpu.async_copy(uemb_hbm.at[uid_v.at[j]], ubufs[j % 2],
                              usems[j % 2])
        cm = pltpu.async_copy(memb_hbm.at[mid_v.at[j]], mbufs[j % 2],
                              msems[j % 2])
        return cu, cm

    pending = start(0)
    for c in bias_copies:
        c.wait()

    def compute(j):
        urows_v = ubufs[j % 2]
        mrows_v = mbufs[j % 2]

        def group(g, _):
            vec = jnp.zeros((L,), jnp.float32)
            for i in range(L):
                r = g * L + i
                acc = urows_v[r, pl.ds(0, L)] * mrows_v[r, pl.ds(0, L)]
                for k in range(1, D // L):
                    acc = acc + (urows_v[r, pl.ds(k * L, L)]
                                 * mrows_v[r, pl.ds(k * L, L)])
                s = jnp.sum(acc)
                vec = jnp.where(lanes == i, s, vec)
            vec = (vec + ub_v[j, pl.ds(g * L, L)] + mb_v[j, pl.ds(g * L, L)])
            out_v[pl.ds(j * CHUNK + g * L, L)] = vec
            return 0

        lax.fori_loop(0, CHUNK // L, group, 0)

    for j in range(NCHUNK):
        cu, cm = pending
        if j + 1 < NCHUNK:
            nxt = start(j + 1)
        cu.wait()
        cm.wait()
        compute(j)
        if j + 1 < NCHUNK:
            pending = nxt

    pltpu.sync_copy(out_v, out_hbm.at[pl.ds(base, BPW)])


@jax.jit
def _run(user_ids, movie_ids, user_emb, movie_emb, user_bias, movie_bias):
    mesh = plsc.VectorSubcoreMesh(core_axis_name="c", subcore_axis_name="s")
    f = functools.partial(
        pl.kernel,
        mesh=mesh,
        compiler_params=pltpu.CompilerParams(needs_layout_passes=False),
        out_type=jax.ShapeDtypeStruct((B,), jnp.float32),
        scratch_types=[
            pltpu.VMEM((NCHUNK, CHUNK), jnp.int32),
            pltpu.VMEM((NCHUNK, CHUNK), jnp.int32),
            pltpu.VMEM((CHUNK, D), jnp.float32),
            pltpu.VMEM((CHUNK, D), jnp.float32),
            pltpu.VMEM((CHUNK, D), jnp.float32),
            pltpu.VMEM((CHUNK, D), jnp.float32),
            pltpu.VMEM((NCHUNK, CHUNK), jnp.float32),
            pltpu.VMEM((NCHUNK, CHUNK), jnp.float32),
            pltpu.VMEM((BPW,), jnp.float32),
            pltpu.SemaphoreType.DMA,
            pltpu.SemaphoreType.DMA,
            pltpu.SemaphoreType.DMA,
            pltpu.SemaphoreType.DMA,
            pltpu.SemaphoreType.DMA,
        ],
    )(_sc_body)
    return f(user_ids, movie_ids, user_emb, movie_emb, user_bias, movie_bias)


def kernel(user_ids, movie_ids, user_emb, movie_emb, user_bias, movie_bias):
    return _run(user_ids.astype(jnp.int32), movie_ids.astype(jnp.int32),
                user_emb, movie_emb,
                user_bias.reshape(-1), movie_bias.reshape(-1))
